# baseline (speedup 1.0000x reference)
import jax
import jax.numpy as jnp
from jax.experimental import pallas as pl
from jax.experimental.pallas import tpu as pltpu

_N = 10000
_HID = 128
_CLS = 64
_BI1 = 200
_BI2 = 400

_F8 = jnp.float8_e4m3fn
_ADJ_SCALE = 2.0 ** 14
_SUP_SCALE = 2.0 ** 8
_INV_SCALE = 1.0 / (_ADJ_SCALE * _SUP_SCALE)


def _l1_kernel(h_ref, w0_ref, b0_ref, adj_ref, x1_ref, adj8_ref, s0_ref):
    i = pl.program_id(0)

    @pl.when(i == 0)
    def _prologue():
        s0_ref[...] = jnp.dot(
            h_ref[...].astype(jnp.bfloat16),
            w0_ref[...].astype(jnp.bfloat16),
            preferred_element_type=jnp.float32,
        ).astype(jnp.bfloat16)

    a = adj_ref[...]
    adj8_ref[...] = (a * _ADJ_SCALE).astype(_F8)
    acc = jnp.dot(a.astype(jnp.bfloat16), s0_ref[...],
                  preferred_element_type=jnp.float32)
    x1_ref[...] = jnp.maximum(acc + b0_ref[...], 0.0).astype(jnp.bfloat16)


def _layer1(h, W0, b0, adj):
    return pl.pallas_call(
        _l1_kernel,
        grid=(_N // _BI1,),
        in_specs=[
            pl.BlockSpec((_N, _HID), lambda i: (0, 0)),
            pl.BlockSpec((_HID, _HID), lambda i: (0, 0)),
            pl.BlockSpec((1, _HID), lambda i: (0, 0)),
            pl.BlockSpec((_BI1, _N), lambda i: (i, 0)),
        ],
        out_specs=[
            pl.BlockSpec((_BI1, _HID), lambda i: (i, 0)),
            pl.BlockSpec((_BI1, _N), lambda i: (i, 0)),
        ],
        out_shape=[
            jax.ShapeDtypeStruct((_N, _HID), jnp.bfloat16),
            jax.ShapeDtypeStruct((_N, _N), _F8),
        ],
        scratch_shapes=[pltpu.VMEM((_N, _HID), jnp.bfloat16)],
    )(h, W0, b0, adj)


def _l2_kernel(x1_ref, w1_ref, b1_ref, wt_ref, wb_ref, bl_ref, adj8_ref,
               o_ref, s1_ref):
    i = pl.program_id(0)

    @pl.when(i == 0)
    def _prologue():
        s1 = jnp.dot(
            x1_ref[...],
            w1_ref[...].astype(jnp.bfloat16),
            preferred_element_type=jnp.float32,
        )
        s1_ref[...] = (s1 * _SUP_SCALE).astype(_F8)

    acc = jnp.dot(adj8_ref[...], s1_ref[...],
                  preferred_element_type=jnp.float32)
    x2 = jnp.maximum(acc * _INV_SCALE + b1_ref[...], 0.0)
    x1_blk = x1_ref[pl.ds(i * _BI2, _BI2), :]
    logits = (
        jnp.dot(x1_blk, wt_ref[...],
                preferred_element_type=jnp.float32)
        + jnp.dot(x2.astype(jnp.bfloat16), wb_ref[...],
                  preferred_element_type=jnp.float32)
        + bl_ref[...]
    )
    shifted = logits - jnp.max(logits, axis=1, keepdims=True)
    lse = jnp.log(jnp.sum(jnp.exp(shifted), axis=1, keepdims=True))
    o_ref[...] = shifted - lse


def _layer2(x1, W1, b1, wl_top, wl_bot, bl, adj8):
    return pl.pallas_call(
        _l2_kernel,
        grid=(_N // _BI2,),
        in_specs=[
            pl.BlockSpec((_N, _HID), lambda i: (0, 0)),
            pl.BlockSpec((_HID, _HID), lambda i: (0, 0)),
            pl.BlockSpec((1, _HID), lambda i: (0, 0)),
            pl.BlockSpec((_HID, _CLS), lambda i: (0, 0)),
            pl.BlockSpec((_HID, _CLS), lambda i: (0, 0)),
            pl.BlockSpec((1, _CLS), lambda i: (0, 0)),
            pl.BlockSpec((_BI2, _N), lambda i: (i, 0)),
        ],
        out_specs=pl.BlockSpec((_BI2, _CLS), lambda i: (i, 0)),
        out_shape=jax.ShapeDtypeStruct((_N, _CLS), jnp.float32),
        scratch_shapes=[pltpu.VMEM((_N, _HID), _F8)],
    )(x1, W1, b1, wl_top, wl_bot, bl, adj8)


def kernel(h, adj, W0, b0, W1, b1, Wl, bl):
    x1, adj8 = _layer1(h, W0, b0.reshape(1, _HID), adj)
    return _layer2(
        x1, W1,
        b1.reshape(1, _HID),
        Wl[:_HID].astype(jnp.bfloat16), Wl[_HID:].astype(jnp.bfloat16),
        bl.reshape(1, _CLS),
        adj8,
    )


# speedup vs baseline: 1.0578x; 1.0578x over previous
import jax
import jax.numpy as jnp
from jax.experimental import pallas as pl
from jax.experimental.pallas import tpu as pltpu

_N = 10000
_HID = 128
_CLS = 64
_BI1 = 400
_BI2 = 1000

_F8 = jnp.float8_e4m3fn
_ADJ_SCALE = 2.0 ** 14
_SUP_SCALE = 2.0 ** 8
_INV_SCALE = 1.0 / (_ADJ_SCALE * _SUP_SCALE)


def _l1_kernel(h_ref, w0_ref, b0_ref, adj_ref, x1_ref, adj8_ref, s0_ref):
    i = pl.program_id(0)

    @pl.when(i == 0)
    def _prologue():
        s0_ref[...] = jnp.dot(
            h_ref[...].astype(jnp.bfloat16),
            w0_ref[...].astype(jnp.bfloat16),
            preferred_element_type=jnp.float32,
        ).astype(jnp.bfloat16)

    a = adj_ref[...]
    adj8_ref[...] = (a * _ADJ_SCALE).astype(_F8)
    acc = jnp.dot(a.astype(jnp.bfloat16), s0_ref[...],
                  preferred_element_type=jnp.float32)
    x1_ref[...] = jnp.maximum(acc + b0_ref[...], 0.0).astype(jnp.bfloat16)


def _layer1(h, W0, b0, adj):
    return pl.pallas_call(
        _l1_kernel,
        grid=(_N // _BI1,),
        in_specs=[
            pl.BlockSpec((_N, _HID), lambda i: (0, 0)),
            pl.BlockSpec((_HID, _HID), lambda i: (0, 0)),
            pl.BlockSpec((1, _HID), lambda i: (0, 0)),
            pl.BlockSpec((_BI1, _N), lambda i: (i, 0)),
        ],
        out_specs=[
            pl.BlockSpec((_BI1, _HID), lambda i: (i, 0)),
            pl.BlockSpec((_BI1, _N), lambda i: (i, 0)),
        ],
        out_shape=[
            jax.ShapeDtypeStruct((_N, _HID), jnp.bfloat16),
            jax.ShapeDtypeStruct((_N, _N), _F8),
        ],
        scratch_shapes=[pltpu.VMEM((_N, _HID), jnp.bfloat16)],
    )(h, W0, b0, adj)


def _l2_kernel(x1_ref, w1_ref, b1_ref, wt_ref, wb_ref, bl_ref, adj8_ref,
               o_ref, s1_ref):
    i = pl.program_id(0)

    @pl.when(i == 0)
    def _prologue():
        s1 = jnp.dot(
            x1_ref[...],
            w1_ref[...].astype(jnp.bfloat16),
            preferred_element_type=jnp.float32,
        )
        s1_ref[...] = (s1 * _SUP_SCALE).astype(_F8)

    acc = jnp.dot(adj8_ref[...], s1_ref[...],
                  preferred_element_type=jnp.float32)
    x2 = jnp.maximum(acc * _INV_SCALE + b1_ref[...], 0.0)
    x1_blk = x1_ref[pl.ds(i * _BI2, _BI2), :]
    logits = (
        jnp.dot(x1_blk, wt_ref[...],
                preferred_element_type=jnp.float32)
        + jnp.dot(x2.astype(jnp.bfloat16), wb_ref[...],
                  preferred_element_type=jnp.float32)
        + bl_ref[...]
    )
    shifted = logits - jnp.max(logits, axis=1, keepdims=True)
    lse = jnp.log(jnp.sum(jnp.exp(shifted), axis=1, keepdims=True))
    o_ref[...] = shifted - lse


def _layer2(x1, W1, b1, wl_top, wl_bot, bl, adj8):
    return pl.pallas_call(
        _l2_kernel,
        grid=(_N // _BI2,),
        in_specs=[
            pl.BlockSpec((_N, _HID), lambda i: (0, 0)),
            pl.BlockSpec((_HID, _HID), lambda i: (0, 0)),
            pl.BlockSpec((1, _HID), lambda i: (0, 0)),
            pl.BlockSpec((_HID, _CLS), lambda i: (0, 0)),
            pl.BlockSpec((_HID, _CLS), lambda i: (0, 0)),
            pl.BlockSpec((1, _CLS), lambda i: (0, 0)),
            pl.BlockSpec((_BI2, _N), lambda i: (i, 0)),
        ],
        out_specs=pl.BlockSpec((_BI2, _CLS), lambda i: (i, 0)),
        out_shape=jax.ShapeDtypeStruct((_N, _CLS), jnp.float32),
        scratch_shapes=[pltpu.VMEM((_N, _HID), _F8)],
    )(x1, W1, b1, wl_top, wl_bot, bl, adj8)


def kernel(h, adj, W0, b0, W1, b1, Wl, bl):
    x1, adj8 = _layer1(h, W0, b0.reshape(1, _HID), adj)
    return _layer2(
        x1, W1,
        b1.reshape(1, _HID),
        Wl[:_HID].astype(jnp.bfloat16), Wl[_HID:].astype(jnp.bfloat16),
        bl.reshape(1, _CLS),
        adj8,
    )
